# baseline (speedup 1.0000x reference)
import functools

import jax
import jax.numpy as jnp
from jax import lax
from jax.experimental import pallas as pl
from jax.experimental.pallas import tpu as pltpu
from jax.experimental.pallas import tpu_sc as plsc

B, S, D = 1024, 200, 64

_info = plsc.get_sparse_core_info()
NC, NS = _info.num_cores, _info.num_subcores
NW = NC * NS
SEQ_W = B // NW
ROWS_W = SEQ_W * S

_SPLIT = 128

_mesh = plsc.VectorSubcoreMesh(core_axis_name="c", subcore_axis_name="s")


@functools.partial(
    pl.kernel,
    out_type=jax.ShapeDtypeStruct((B * S, D), jnp.float32),
    mesh=_mesh,
    compiler_params=pltpu.CompilerParams(use_tc_tiling_on_sc=False),
    scratch_types=[
        pltpu.VMEM((ROWS_W,), jnp.int32),
        pltpu.VMEM((S, D), jnp.float32),
        pltpu.VMEM((S, D), jnp.float32),
        pltpu.VMEM((S, D), jnp.float32),
        pltpu.SemaphoreType.DMA,
        pltpu.SemaphoreType.DMA,
        pltpu.SemaphoreType.DMA,
        pltpu.SemaphoreType.DMA,
    ],
)
def _embed(x_hbm, tok_hbm, pos_hbm, out_hbm, idx_v, pos_v, buf0, buf1,
           gsem0, gsem1, ssem0, ssem1):
    wid = lax.axis_index("s") * NC + lax.axis_index("c")
    base = wid * ROWS_W

    pltpu.sync_copy(pos_hbm, pos_v)
    pltpu.sync_copy(x_hbm.at[pl.ds(base, ROWS_W)], idx_v)

    bufs = (buf0, buf1)
    gsems = (gsem0, gsem1)
    ssems = (ssem0, ssem1)

    def start_gather(s):
        b = s % 2
        d0 = pltpu.async_copy(
            tok_hbm.at[idx_v.at[pl.ds(s * S, _SPLIT)]],
            bufs[b].at[pl.ds(0, _SPLIT)], gsems[b])
        d1 = pltpu.async_copy(
            tok_hbm.at[idx_v.at[pl.ds(s * S + _SPLIT, S - _SPLIT)]],
            bufs[b].at[pl.ds(_SPLIT, S - _SPLIT)], gsems[b])
        return (d0, d1)

    def add_pos(buf):
        def body(r, carry):
            for k in range(D // 16):
                sl = pl.ds(k * 16, 16)
                plsc.addupdate(buf.at[r, sl], pos_v[r, sl])
            return carry
        lax.fori_loop(0, S, body, 0)

    gd = [None, None]
    sd = [None, None]
    for s in range(SEQ_W + 1):
        if s < SEQ_W:
            b = s % 2
            if sd[b] is not None:
                sd[b].wait()
            gd[b] = start_gather(s)
        if s >= 1:
            sp = s - 1
            bp = sp % 2
            for d in gd[bp]:
                d.wait()
            add_pos(bufs[bp])
            sd[bp] = pltpu.async_copy(
                bufs[bp], out_hbm.at[pl.ds(base + sp * S, S)], ssems[bp])
    sd[(SEQ_W - 2) % 2].wait()
    sd[(SEQ_W - 1) % 2].wait()


def kernel(x, token_table, pos_table):
    xf = x.reshape(B * S).astype(jnp.int32)
    out = _embed(xf, token_table, pos_table)
    return out.reshape(B, S, D)
